# baseline (speedup 1.0000x reference)
import functools

import jax
import jax.numpy as jnp
from jax.experimental import pallas as pl
from jax.experimental.pallas import tpu as pltpu

N_TOKENS = 2048
D_MODEL = 1024
N_EXPERTS = 8
BT = 256
NBMAX = 23
M_ROWS = NBMAX * BT


def _router_body(x_ref, gw_ref, gb_ref,
                 pos0_ref, pos1_ref, w0_ref, w1_ref, be_ref, nb_ref):
    lT = jax.lax.dot_general(
        gw_ref[...], x_ref[...], (((1,), (1,)), ((), ())),
        preferred_element_type=jnp.float32) + gb_ref[...]
    eidx = jax.lax.broadcasted_iota(jnp.int32, (N_EXPERTS, N_TOKENS), 0)
    m1 = jnp.max(lT, axis=0, keepdims=True)
    i1 = jnp.min(jnp.where(lT == m1, eidx, N_EXPERTS), axis=0, keepdims=True)
    masked = jnp.where(eidx == i1, -jnp.inf, lT)
    m2 = jnp.max(masked, axis=0, keepdims=True)
    i2 = jnp.min(jnp.where(masked == m2, eidx, N_EXPERTS), axis=0,
                 keepdims=True)
    t = jnp.exp(m2 - m1)
    w0_ref[...] = 1.0 / (1.0 + t)
    w1_ref[...] = t / (1.0 + t)

    sel = ((eidx == i1) | (eidx == i2)).astype(jnp.int32)
    run = sel
    k = 1
    while k < N_TOKENS:
        shifted = jnp.concatenate(
            [jnp.zeros((N_EXPERTS, k), jnp.int32), run[:, :N_TOKENS - k]],
            axis=1)
        run = run + shifted
        k *= 2
    rank = run - sel
    counts = run[:, N_TOKENS - 1:N_TOKENS]
    nblk = (counts + (BT - 1)) // BT
    padded = nblk * BT
    lo = (jax.lax.broadcasted_iota(jnp.int32, (N_EXPERTS, N_EXPERTS), 0)
          > jax.lax.broadcasted_iota(jnp.int32, (N_EXPERTS, N_EXPERTS), 1)
          ).astype(jnp.float32)
    P = jax.lax.dot_general(
        lo, padded.astype(jnp.float32), (((1,), (0,)), ((), ())),
        preferred_element_type=jnp.float32).astype(jnp.int32)
    pos = P + rank
    pos0_ref[...] = jnp.sum(jnp.where(eidx == i1, pos, 0), axis=0,
                            keepdims=True)
    pos1_ref[...] = jnp.sum(jnp.where(eidx == i2, pos, 0), axis=0,
                            keepdims=True)
    Pb = P // BT
    bidx = jax.lax.broadcasted_iota(jnp.int32, (N_EXPERTS, NBMAX), 1)
    be_ref[...] = (jnp.sum((bidx >= Pb).astype(jnp.int32), axis=0,
                           keepdims=True) - 1)
    nb_ref[...] = jnp.sum(nblk, axis=0, keepdims=True)


def _router(x, gate_W, gate_b):
    out_shapes = (
        jax.ShapeDtypeStruct((1, N_TOKENS), jnp.int32),
        jax.ShapeDtypeStruct((1, N_TOKENS), jnp.int32),
        jax.ShapeDtypeStruct((1, N_TOKENS), jnp.float32),
        jax.ShapeDtypeStruct((1, N_TOKENS), jnp.float32),
        jax.ShapeDtypeStruct((1, NBMAX), jnp.int32),
        jax.ShapeDtypeStruct((1, 1), jnp.int32),
    )
    return pl.pallas_call(
        _router_body,
        grid=(1,),
        in_specs=[
            pl.BlockSpec((N_TOKENS, D_MODEL), lambda i: (0, 0)),
            pl.BlockSpec((N_EXPERTS, D_MODEL), lambda i: (0, 0)),
            pl.BlockSpec((N_EXPERTS, 1), lambda i: (0, 0)),
        ],
        out_specs=tuple(
            pl.BlockSpec(s.shape, lambda i: (0,) * len(s.shape))
            for s in out_shapes),
        out_shape=out_shapes,
    )(x, gate_W, gate_b.reshape(N_EXPERTS, 1))


def _grouped_body(be_ref, nb_ref, disp_ref, W1_ref, b1_ref, W2_ref, b2_ref,
                  wgt_ref, y_ref):
    i = pl.program_id(0)

    @pl.when(i < nb_ref[0])
    def _():
        xs = disp_ref[...]
        h = jnp.maximum(
            jnp.dot(xs, W1_ref[0], preferred_element_type=jnp.float32)
            + b1_ref[0], 0.0)
        o = (jnp.dot(h.astype(jnp.bfloat16), W2_ref[0],
                     preferred_element_type=jnp.float32) + b2_ref[0])
        y_ref[...] = o * wgt_ref[0]


def _grouped_matmul(be, nb, disp, W1b, b1r, W2b, b2r, wgt3d):
    grid_spec = pltpu.PrefetchScalarGridSpec(
        num_scalar_prefetch=2,
        grid=(NBMAX,),
        in_specs=[
            pl.BlockSpec((BT, D_MODEL), lambda i, be, nb: (i, 0)),
            pl.BlockSpec((1, D_MODEL, D_MODEL),
                         lambda i, be, nb: (be[i], 0, 0)),
            pl.BlockSpec((1, 1, D_MODEL), lambda i, be, nb: (be[i], 0, 0)),
            pl.BlockSpec((1, D_MODEL, D_MODEL),
                         lambda i, be, nb: (be[i], 0, 0)),
            pl.BlockSpec((1, 1, D_MODEL), lambda i, be, nb: (be[i], 0, 0)),
            pl.BlockSpec((1, BT, 1), lambda i, be, nb: (i, 0, 0)),
        ],
        out_specs=pl.BlockSpec((BT, D_MODEL), lambda i, be, nb: (i, 0)),
    )
    return pl.pallas_call(
        _grouped_body,
        grid_spec=grid_spec,
        out_shape=jax.ShapeDtypeStruct((M_ROWS, D_MODEL), jnp.float32),
    )(be, nb, disp, W1b, b1r, W2b, b2r, wgt3d)


@jax.jit
def kernel(x, gate_W, gate_b, W1, b1, W2, b2):
    xb = x.astype(jnp.bfloat16)
    W1b = W1.astype(jnp.bfloat16)
    W2b = W2.astype(jnp.bfloat16)
    b1r = b1.reshape(N_EXPERTS, 1, D_MODEL)
    b2r = b2.reshape(N_EXPERTS, 1, D_MODEL)

    pos0, pos1, w0, w1, be, nb = _router(x, gate_W, gate_b)
    pos0f, pos1f = pos0.reshape(-1), pos1.reshape(-1)

    ar = jnp.arange(N_TOKENS, dtype=jnp.int32)
    tok = (jnp.zeros((M_ROWS,), jnp.int32)
           .at[pos0f].set(ar).at[pos1f].set(ar))
    wgt = (jnp.zeros((M_ROWS,), jnp.float32)
           .at[pos0f].set(w0.reshape(-1)).at[pos1f].set(w1.reshape(-1)))
    disp = jnp.take(xb, tok, axis=0)

    y = _grouped_matmul(be.reshape(-1), nb.reshape(-1), disp,
                        W1b, b1r, W2b, b2r, wgt.reshape(NBMAX, BT, 1))

    out = jnp.take(y, pos0f, axis=0) + jnp.take(y, pos1f, axis=0)
    return out
